# Initial kernel scaffold; baseline (speedup 1.0000x reference)
import jax
import jax.numpy as jnp
from jax.experimental import pallas as pl


def kernel(x, enc1_w, enc1_b, bn1_g, bn1_b, enc2_w, enc2_b, bn2_g, bn2_b, pre_w, pre_b, emb, post_w, post_b, dec1_w, dec1_b, bn3_g, bn3_b, dec2_w, dec2_b):
    raise NotImplementedError("write your pallas kernel here")


# speedup vs baseline: 1.7168x; 1.7168x over previous
import jax
import jax.numpy as jnp
from jax.experimental import pallas as pl

_EPS = 1e-5


def _conv2d(x, w, b, stride, padding):
    out = jax.lax.conv_general_dilated(
        x, w, (stride, stride), ((padding, padding), (padding, padding)),
        dimension_numbers=('NCHW', 'OIHW', 'NCHW'))
    return out + b[None, :, None, None]


def _conv_transpose2d(x, w, b, stride, padding):
    k = jnp.transpose(w, (1, 0, 2, 3))[:, :, ::-1, ::-1]
    pad = w.shape[2] - 1 - padding
    out = jax.lax.conv_general_dilated(
        x, k, (1, 1), ((pad, pad), (pad, pad)), lhs_dilation=(stride, stride),
        dimension_numbers=('NCHW', 'OIHW', 'NCHW'))
    return out + b[None, :, None, None]


def _bn_relu(x, gamma, beta):
    mean = jnp.mean(x, axis=(0, 2, 3), keepdims=True)
    var = jnp.mean((x - mean) ** 2, axis=(0, 2, 3), keepdims=True)
    xn = (x - mean) / jnp.sqrt(var + _EPS)
    return jax.nn.relu(xn * gamma[None, :, None, None] + beta[None, :, None, None])


def _vq_kernel(c2_ref, scale_ref, shift_ref, wpre_ref, bpre_ref, emb_ref,
               wpost_ref, bpost_ref, out_ref, loss_ref):
    h = c2_ref[0]
    h = jnp.maximum(h * scale_ref[...] + shift_ref[...], 0.0)
    q = jnp.dot(wpre_ref[...], h,
                preferred_element_type=jnp.float32) + bpre_ref[...]
    emb = emb_ref[...]
    cross = jnp.dot(emb, q, preferred_element_type=jnp.float32)
    qsq = jnp.sum(q * q, axis=0, keepdims=True)
    esq = jnp.sum(emb * emb, axis=1, keepdims=True)
    d2 = qsq - 2.0 * cross + esq
    dist = jnp.sqrt(jnp.maximum(d2, 0.0))
    d0, d1, d2_ = dist[0:1, :], dist[1:2, :], dist[2:3, :]
    c1 = d1 < d0
    b01 = jnp.where(c1, d1, d0)
    c2sel = d2_ < b01
    sel2 = c2sel
    sel1 = jnp.logical_and(jnp.logical_not(c2sel), c1)
    sel0 = jnp.logical_and(jnp.logical_not(c2sel), jnp.logical_not(c1))
    onehot = jnp.concatenate(
        [sel0.astype(jnp.float32), sel1.astype(jnp.float32),
         sel2.astype(jnp.float32)], axis=0)
    zq = jnp.dot(emb.T, onehot, preferred_element_type=jnp.float32)
    loss_ref[...] = jnp.sum((zq - q) ** 2)[None, None, None]
    out = jnp.dot(wpost_ref[...], zq,
                  preferred_element_type=jnp.float32) + bpost_ref[...]
    out_ref[0] = out


def kernel(x, enc1_w, enc1_b, bn1_g, bn1_b, enc2_w, enc2_b, bn2_g, bn2_b,
           pre_w, pre_b, emb, post_w, post_b, dec1_w, dec1_b, bn3_g, bn3_b,
           dec2_w, dec2_b):
    h1 = _bn_relu(_conv2d(x, enc1_w, enc1_b, 2, 1), bn1_g, bn1_b)
    c2 = _conv2d(h1, enc2_w, enc2_b, 2, 2 // 2)
    bs, cc, hh, ww = c2.shape
    P = hh * ww

    mean = jnp.mean(c2, axis=(0, 2, 3))
    var = jnp.mean((c2 - mean[None, :, None, None]) ** 2, axis=(0, 2, 3))
    scale = (bn2_g / jnp.sqrt(var + _EPS))[:, None]
    shift = (bn2_b - mean * scale[:, 0])[:, None]

    c2r = c2.reshape(bs, cc, P)
    wpre = pre_w.reshape(2, cc)
    bpre = pre_b[:, None]
    wpost = post_w.reshape(cc, 2)
    bpost = post_b[:, None]

    small = lambda shp: pl.BlockSpec(shp, lambda b: (0, 0))
    di_r, loss_parts = pl.pallas_call(
        _vq_kernel,
        grid=(bs,),
        in_specs=[
            pl.BlockSpec((1, cc, P), lambda b: (b, 0, 0)),
            small((cc, 1)), small((cc, 1)),
            small((2, cc)), small((2, 1)),
            small((3, 2)),
            small((cc, 2)), small((cc, 1)),
        ],
        out_specs=[
            pl.BlockSpec((1, cc, P), lambda b: (b, 0, 0)),
            pl.BlockSpec((1, 1, 1), lambda b: (b, 0, 0)),
        ],
        out_shape=[
            jax.ShapeDtypeStruct((bs, cc, P), jnp.float32),
            jax.ShapeDtypeStruct((bs, 1, 1), jnp.float32),
        ],
    )(c2r, scale, shift, wpre, bpre, emb, wpost, bpost)

    loss = jnp.sum(loss_parts) * 1.25 / (bs * 2 * P)
    di = di_r.reshape(bs, cc, hh, ww)

    h3 = _bn_relu(_conv_transpose2d(di, dec1_w, dec1_b, 2, 1), bn3_g, bn3_b)
    decoded = jax.nn.sigmoid(_conv_transpose2d(h3, dec2_w, dec2_b, 2, 1))
    return (decoded, loss)
